# Initial kernel scaffold; baseline (speedup 1.0000x reference)
import jax
import jax.numpy as jnp
import numpy as np
from jax.experimental import pallas as pl
from jax.experimental.pallas import tpu as pltpu

_BS = 2
_Q = 300
_C = 42
_T = 4
_HW = 48
_K = 100
_FLAT = _Q * _C
_PAD_COLS = 128
_PAD_ROWS = 99
_OUT = 256
_CROP_H = 180
_CROP_W = 192


def _interp_matrices():
    i = np.arange(_OUT)
    r = np.floor((i + 0.5) * (_CROP_H / _OUT)).astype(np.int64)
    y = (r + 0.5) / 4.0 - 0.5
    j0 = np.floor(y).astype(np.int64)
    w1 = (y - j0).astype(np.float32)
    ry = np.zeros((_OUT, _HW), np.float32)
    np.add.at(ry, (i, np.clip(j0, 0, _HW - 1)), 1.0 - w1)
    np.add.at(ry, (i, np.clip(j0 + 1, 0, _HW - 1)), w1)

    j = np.arange(_OUT)
    c = np.floor((j + 0.5) * (_CROP_W / _OUT)).astype(np.int64)
    x = (c + 0.5) / 4.0 - 0.5
    c0 = np.floor(x).astype(np.int64)
    wx = (x - c0).astype(np.float32)
    cx = np.zeros((_HW, _OUT), np.float32)
    np.add.at(cx, (np.clip(c0, 0, _HW - 1), j), 1.0 - wx)
    np.add.at(cx, (np.clip(c0 + 1, 0, _HW - 1), j), wx)
    return ry, cx


_RY, _CX = _interp_matrices()


def _topk_body(prob_ref, scores_ref, boxes_ref, labels_ref):
    prob = prob_ref[0]
    rows = jax.lax.broadcasted_iota(jnp.int32, (_PAD_ROWS, _PAD_COLS), 0)
    cols = jax.lax.broadcasted_iota(jnp.int32, (_PAD_ROWS, _PAD_COLS), 1)
    flat = rows * _PAD_COLS + cols
    lanes = jax.lax.broadcasted_iota(jnp.int32, (1, _PAD_COLS), 1)

    def body(k, carry):
        p, sv, iv = carry
        m = jnp.max(p)
        idx = jnp.min(jnp.where(p == m, flat, jnp.int32(1 << 30)))
        sv = jnp.where(lanes == k, m, sv)
        iv = jnp.where(lanes == k, idx, iv)
        p = jnp.where(flat == idx, jnp.float32(-1.0), p)
        return p, sv, iv

    init = (prob,
            jnp.zeros((1, _PAD_COLS), jnp.float32),
            jnp.zeros((1, _PAD_COLS), jnp.int32))
    _, sv, iv = jax.lax.fori_loop(0, _K, body, init)
    scores_ref[0] = sv
    boxes_ref[0] = iv // _C
    labels_ref[0] = iv % _C


def _mask_body(boxes_smem, mask_ref, ry_ref, cx_ref, refs_in_ref,
               masks_out_ref, refs_out_ref):
    del boxes_smem
    ry = ry_ref[...]
    cx = cx_ref[...]
    for t in range(_T):
        mt = mask_ref[0, 0, t]
        a = jnp.dot(ry, mt, preferred_element_type=jnp.float32)
        o = jnp.dot(a, cx, preferred_element_type=jnp.float32)
        masks_out_ref[0, 0, t] = (o > 0).astype(jnp.uint8)
    refs_out_ref[...] = refs_in_ref[...]


@jax.jit
def kernel(pred_logits, pred_masks, reference_points, orig_target_sizes,
           max_target_sizes):
    del orig_target_sizes, max_target_sizes
    prob = jax.nn.sigmoid(pred_logits).reshape(_BS, _FLAT)
    prob = jnp.pad(prob, ((0, 0), (0, _PAD_ROWS * _PAD_COLS - _FLAT)),
                   constant_values=-1.0)
    prob = prob.reshape(_BS, _PAD_ROWS, _PAD_COLS)

    scores3, boxes3, labels3 = pl.pallas_call(
        _topk_body,
        grid=(_BS,),
        in_specs=[pl.BlockSpec((1, _PAD_ROWS, _PAD_COLS), lambda b: (b, 0, 0))],
        out_specs=[
            pl.BlockSpec((1, 1, _PAD_COLS), lambda b: (b, 0, 0)),
            pl.BlockSpec((1, 1, _PAD_COLS), lambda b: (b, 0, 0)),
            pl.BlockSpec((1, 1, _PAD_COLS), lambda b: (b, 0, 0)),
        ],
        out_shape=[
            jax.ShapeDtypeStruct((_BS, 1, _PAD_COLS), jnp.float32),
            jax.ShapeDtypeStruct((_BS, 1, _PAD_COLS), jnp.int32),
            jax.ShapeDtypeStruct((_BS, 1, _PAD_COLS), jnp.int32),
        ],
    )(prob)

    scores = scores3.reshape(_BS, _PAD_COLS)[:, :_K]
    boxes = boxes3.reshape(_BS, _PAD_COLS)
    labels = labels3.reshape(_BS, _PAD_COLS)[:, :_K]

    refs_in = reference_points.reshape(_BS, _Q, 1, 4)
    grid_spec = pltpu.PrefetchScalarGridSpec(
        num_scalar_prefetch=1,
        grid=(_BS, _K),
        in_specs=[
            pl.BlockSpec((1, 1, _T, _HW, _HW),
                         lambda b, k, bx: (b, bx[b, k], 0, 0, 0)),
            pl.BlockSpec((_OUT, _HW), lambda b, k, bx: (0, 0)),
            pl.BlockSpec((_HW, _OUT), lambda b, k, bx: (0, 0)),
            pl.BlockSpec((1, 1, 1, 4), lambda b, k, bx: (b, bx[b, k], 0, 0)),
        ],
        out_specs=[
            pl.BlockSpec((1, 1, _T, _OUT, _OUT), lambda b, k, bx: (b, k, 0, 0, 0)),
            pl.BlockSpec((1, 1, 1, 4), lambda b, k, bx: (b, k, 0, 0)),
        ],
    )
    masks, refs4 = pl.pallas_call(
        _mask_body,
        grid_spec=grid_spec,
        out_shape=[
            jax.ShapeDtypeStruct((_BS, _K, _T, _OUT, _OUT), jnp.uint8),
            jax.ShapeDtypeStruct((_BS, _K, 1, 4), jnp.float32),
        ],
    )(boxes, pred_masks, jnp.asarray(_RY), jnp.asarray(_CX), refs_in)

    refs = refs4.reshape(_BS, _K, 4)
    return (scores, labels, refs, masks)


# speedup vs baseline: 2.5564x; 2.5564x over previous
import jax
import jax.numpy as jnp
import numpy as np
from jax.experimental import pallas as pl
from jax.experimental.pallas import tpu as pltpu

_BS = 2
_Q = 300
_C = 42
_T = 4
_HW = 48
_K = 100
_FLAT = _Q * _C
_PAD_COLS = 128
_PAD_ROWS = 99
_OUT = 256
_CROP_H = 180
_CROP_W = 192


def _interp_matrices():
    i = np.arange(_OUT)
    r = np.floor((i + 0.5) * (_CROP_H / _OUT)).astype(np.int64)
    y = (r + 0.5) / 4.0 - 0.5
    j0 = np.floor(y).astype(np.int64)
    w1 = (y - j0).astype(np.float32)
    ry = np.zeros((_OUT, _HW), np.float32)
    np.add.at(ry, (i, np.clip(j0, 0, _HW - 1)), 1.0 - w1)
    np.add.at(ry, (i, np.clip(j0 + 1, 0, _HW - 1)), w1)

    j = np.arange(_OUT)
    c = np.floor((j + 0.5) * (_CROP_W / _OUT)).astype(np.int64)
    x = (c + 0.5) / 4.0 - 0.5
    c0 = np.floor(x).astype(np.int64)
    wx = (x - c0).astype(np.float32)
    cx = np.zeros((_HW, _OUT), np.float32)
    np.add.at(cx, (np.clip(c0, 0, _HW - 1), j), 1.0 - wx)
    np.add.at(cx, (np.clip(c0 + 1, 0, _HW - 1), j), wx)
    return ry, cx


_RY, _CX = _interp_matrices()


def _topk_body(prob_ref, scores_ref, boxes_ref, labels_ref):
    prob = prob_ref[0]
    rows = jax.lax.broadcasted_iota(jnp.int32, (_PAD_ROWS, _PAD_COLS), 0)
    cols = jax.lax.broadcasted_iota(jnp.int32, (_PAD_ROWS, _PAD_COLS), 1)
    flat = rows * _PAD_COLS + cols
    lanes = jax.lax.broadcasted_iota(jnp.int32, (1, _PAD_COLS), 1)

    def body(k, carry):
        p, sv, iv = carry
        m = jnp.max(p)
        idx = jnp.min(jnp.where(p == m, flat, jnp.int32(1 << 30)))
        sv = jnp.where(lanes == k, m, sv)
        iv = jnp.where(lanes == k, idx, iv)
        p = jnp.where(flat == idx, jnp.float32(-1.0), p)
        return p, sv, iv

    init = (prob,
            jnp.zeros((1, _PAD_COLS), jnp.float32),
            jnp.zeros((1, _PAD_COLS), jnp.int32))
    _, sv, iv = jax.lax.fori_loop(0, _K, body, init)
    scores_ref[0] = sv
    boxes_ref[0] = iv // _C
    labels_ref[0] = iv % _C


def _mask_body(boxes_smem, mask_ref, ry_ref, cx_ref, refs_in_ref,
               masks_out_ref, refs_out_ref):
    del boxes_smem
    ry = ry_ref[...]
    cx = cx_ref[...]
    for t in range(_T):
        mt = mask_ref[0, 0, t]
        a = jnp.dot(ry, mt, preferred_element_type=jnp.float32,
                    precision=jax.lax.Precision.HIGHEST)
        o = jnp.dot(a, cx, preferred_element_type=jnp.float32,
                    precision=jax.lax.Precision.HIGHEST)
        masks_out_ref[0, 0, t] = (o > 0).astype(jnp.uint8)
    refs_out_ref[...] = refs_in_ref[...]


@jax.jit
def kernel(pred_logits, pred_masks, reference_points, orig_target_sizes,
           max_target_sizes):
    del orig_target_sizes, max_target_sizes
    prob = jax.nn.sigmoid(pred_logits).reshape(_BS, _FLAT)
    prob = jnp.pad(prob, ((0, 0), (0, _PAD_ROWS * _PAD_COLS - _FLAT)),
                   constant_values=-1.0)
    prob = prob.reshape(_BS, _PAD_ROWS, _PAD_COLS)

    scores3, boxes3, labels3 = pl.pallas_call(
        _topk_body,
        grid=(_BS,),
        in_specs=[pl.BlockSpec((1, _PAD_ROWS, _PAD_COLS), lambda b: (b, 0, 0))],
        out_specs=[
            pl.BlockSpec((1, 1, _PAD_COLS), lambda b: (b, 0, 0)),
            pl.BlockSpec((1, 1, _PAD_COLS), lambda b: (b, 0, 0)),
            pl.BlockSpec((1, 1, _PAD_COLS), lambda b: (b, 0, 0)),
        ],
        out_shape=[
            jax.ShapeDtypeStruct((_BS, 1, _PAD_COLS), jnp.float32),
            jax.ShapeDtypeStruct((_BS, 1, _PAD_COLS), jnp.int32),
            jax.ShapeDtypeStruct((_BS, 1, _PAD_COLS), jnp.int32),
        ],
    )(prob)

    scores = scores3.reshape(_BS, _PAD_COLS)[:, :_K]
    boxes = boxes3.reshape(_BS, _PAD_COLS)
    labels = labels3.reshape(_BS, _PAD_COLS)[:, :_K]

    refs_in = reference_points.reshape(_BS, _Q, 1, 4)
    grid_spec = pltpu.PrefetchScalarGridSpec(
        num_scalar_prefetch=1,
        grid=(_BS, _K),
        in_specs=[
            pl.BlockSpec((1, 1, _T, _HW, _HW),
                         lambda b, k, bx: (b, bx[b, k], 0, 0, 0)),
            pl.BlockSpec((_OUT, _HW), lambda b, k, bx: (0, 0)),
            pl.BlockSpec((_HW, _OUT), lambda b, k, bx: (0, 0)),
            pl.BlockSpec((1, 1, 1, 4), lambda b, k, bx: (b, bx[b, k], 0, 0)),
        ],
        out_specs=[
            pl.BlockSpec((1, 1, _T, _OUT, _OUT), lambda b, k, bx: (b, k, 0, 0, 0)),
            pl.BlockSpec((1, 1, 1, 4), lambda b, k, bx: (b, k, 0, 0)),
        ],
    )
    masks, refs4 = pl.pallas_call(
        _mask_body,
        grid_spec=grid_spec,
        out_shape=[
            jax.ShapeDtypeStruct((_BS, _K, _T, _OUT, _OUT), jnp.uint8),
            jax.ShapeDtypeStruct((_BS, _K, 1, 4), jnp.float32),
        ],
    )(boxes, pred_masks, jnp.asarray(_RY), jnp.asarray(_CX), refs_in)

    refs = refs4.reshape(_BS, _K, 4)
    return (scores, labels, refs, masks)


# speedup vs baseline: 5.8647x; 2.2941x over previous
import jax
import jax.numpy as jnp
import numpy as np
from jax.experimental import pallas as pl
from jax.experimental.pallas import tpu as pltpu

_BS = 2
_Q = 300
_QP = 304
_C = 42
_T = 4
_HW = 48
_K = 100
_G = 10
_FLAT = _Q * _C
_PAD_COLS = 128
_PAD_ROWS = 99
_OUT = 256
_CROP_H = 180
_CROP_W = 192


def _interp_matrices():
    i = np.arange(_OUT)
    r = np.floor((i + 0.5) * (_CROP_H / _OUT)).astype(np.int64)
    y = (r + 0.5) / 4.0 - 0.5
    j0 = np.floor(y).astype(np.int64)
    w1 = (y - j0).astype(np.float32)
    ry = np.zeros((_OUT, _HW), np.float32)
    np.add.at(ry, (i, np.clip(j0, 0, _HW - 1)), 1.0 - w1)
    np.add.at(ry, (i, np.clip(j0 + 1, 0, _HW - 1)), w1)

    j = np.arange(_OUT)
    c = np.floor((j + 0.5) * (_CROP_W / _OUT)).astype(np.int64)
    x = (c + 0.5) / 4.0 - 0.5
    c0 = np.floor(x).astype(np.int64)
    wx = (x - c0).astype(np.float32)
    cx = np.zeros((_HW, _OUT), np.float32)
    np.add.at(cx, (np.clip(c0, 0, _HW - 1), j), 1.0 - wx)
    np.add.at(cx, (np.clip(c0 + 1, 0, _HW - 1), j), wx)
    return ry, cx


_RY, _CX = _interp_matrices()


def _topk_body(prob_ref, refs_ref, scores_ref, boxes_ref, labels_ref,
               refs_out_ref):
    rows = jax.lax.broadcasted_iota(jnp.int32, (_PAD_ROWS, _PAD_COLS), 0)
    cols = jax.lax.broadcasted_iota(jnp.int32, (_PAD_ROWS, _PAD_COLS), 1)
    flat = rows * _PAD_COLS + cols
    lanes = jax.lax.broadcasted_iota(jnp.int32, (1, _PAD_COLS), 1)

    def body(k, carry):
        p0, p1, sv0, sv1, iv0, iv1 = carry
        m0 = jnp.max(p0)
        m1 = jnp.max(p1)
        idx0 = jnp.min(jnp.where(p0 == m0, flat, jnp.int32(1 << 30)))
        idx1 = jnp.min(jnp.where(p1 == m1, flat, jnp.int32(1 << 30)))
        sv0 = jnp.where(lanes == k, m0, sv0)
        sv1 = jnp.where(lanes == k, m1, sv1)
        iv0 = jnp.where(lanes == k, idx0, iv0)
        iv1 = jnp.where(lanes == k, idx1, iv1)
        p0 = jnp.where(flat == idx0, jnp.float32(-1.0), p0)
        p1 = jnp.where(flat == idx1, jnp.float32(-1.0), p1)
        return p0, p1, sv0, sv1, iv0, iv1

    zf = jnp.zeros((1, _PAD_COLS), jnp.float32)
    zi = jnp.zeros((1, _PAD_COLS), jnp.int32)
    init = (prob_ref[0], prob_ref[1], zf, zf, zi, zi)
    _, _, sv0, sv1, iv0, iv1 = jax.lax.fori_loop(0, _K, body, init)

    qrow = jax.lax.broadcasted_iota(jnp.int32, (_QP, _PAD_COLS), 0)
    for b, (sv, iv) in enumerate(((sv0, iv0), (sv1, iv1))):
        bx = iv // _C
        scores_ref[b] = sv
        boxes_ref[b] = bx
        labels_ref[b] = iv % _C
        oht = (qrow == jnp.broadcast_to(bx, (_QP, _PAD_COLS))).astype(
            jnp.float32)
        refs_out_ref[b] = jax.lax.dot_general(
            oht, refs_ref[b], (((0,), (0,)), ((), ())),
            preferred_element_type=jnp.float32)


def _mask_body(boxes_smem, *args):
    del boxes_smem
    mrefs = args[:_G]
    ry2_ref, cx_ref, masks_out_ref = args[_G], args[_G + 1], args[_G + 2]
    ry2 = ry2_ref[...]
    cx = cx_ref[...]
    for g, mref in enumerate(mrefs):
        m2 = mref[0, 0]
        hi = m2.astype(jnp.bfloat16)
        lo = (m2 - hi.astype(jnp.float32)).astype(jnp.bfloat16)
        a1 = jnp.concatenate([hi, lo], axis=0)
        c2p = jnp.dot(a1, cx, preferred_element_type=jnp.float32)
        c2 = c2p[:_T * _HW] + c2p[_T * _HW:]
        c2h = c2.astype(jnp.bfloat16)
        c2l = (c2 - c2h.astype(jnp.float32)).astype(jnp.bfloat16)
        for t in range(_T):
            b2 = jnp.concatenate([c2h[t * _HW:(t + 1) * _HW],
                                  c2l[t * _HW:(t + 1) * _HW]], axis=0)
            o = jnp.dot(ry2, b2, preferred_element_type=jnp.float32)
            masks_out_ref[0, g, t] = (o > 0).astype(jnp.uint8)


@jax.jit
def kernel(pred_logits, pred_masks, reference_points, orig_target_sizes,
           max_target_sizes):
    del orig_target_sizes, max_target_sizes
    prob = jax.nn.sigmoid(pred_logits).reshape(_BS, _FLAT)
    prob = jnp.pad(prob, ((0, 0), (0, _PAD_ROWS * _PAD_COLS - _FLAT)),
                   constant_values=-1.0)
    prob = prob.reshape(_BS, _PAD_ROWS, _PAD_COLS)
    refs_pad = jnp.pad(reference_points, ((0, 0), (0, _QP - _Q), (0, 0)))

    scores3, boxes3, labels3, refsg = pl.pallas_call(
        _topk_body,
        grid=(1,),
        in_specs=[
            pl.BlockSpec((_BS, _PAD_ROWS, _PAD_COLS), lambda i: (0, 0, 0)),
            pl.BlockSpec((_BS, _QP, 4), lambda i: (0, 0, 0)),
        ],
        out_specs=[
            pl.BlockSpec((_BS, 1, _PAD_COLS), lambda i: (0, 0, 0)),
            pl.BlockSpec((_BS, 1, _PAD_COLS), lambda i: (0, 0, 0)),
            pl.BlockSpec((_BS, 1, _PAD_COLS), lambda i: (0, 0, 0)),
            pl.BlockSpec((_BS, _PAD_COLS, 4), lambda i: (0, 0, 0)),
        ],
        out_shape=[
            jax.ShapeDtypeStruct((_BS, 1, _PAD_COLS), jnp.float32),
            jax.ShapeDtypeStruct((_BS, 1, _PAD_COLS), jnp.int32),
            jax.ShapeDtypeStruct((_BS, 1, _PAD_COLS), jnp.int32),
            jax.ShapeDtypeStruct((_BS, _PAD_COLS, 4), jnp.float32),
        ],
    )(prob, refs_pad)

    scores = scores3.reshape(_BS, _PAD_COLS)[:, :_K]
    boxes = boxes3.reshape(_BS, _PAD_COLS)
    labels = labels3.reshape(_BS, _PAD_COLS)[:, :_K]
    refs = refsg[:, :_K, :]

    pred_masks_r = pred_masks.reshape(_BS, _Q, _T * _HW, _HW)
    grid_spec = pltpu.PrefetchScalarGridSpec(
        num_scalar_prefetch=1,
        grid=(_BS, _K // _G),
        in_specs=[
            pl.BlockSpec((1, 1, _T * _HW, _HW),
                         (lambda b, k, bx, g=g: (b, bx[b, _G * k + g], 0, 0)))
            for g in range(_G)
        ] + [
            pl.BlockSpec((_OUT, 2 * _HW), lambda b, k, bx: (0, 0)),
            pl.BlockSpec((_HW, _OUT), lambda b, k, bx: (0, 0)),
        ],
        out_specs=[
            pl.BlockSpec((1, _G, _T, _OUT, _OUT),
                         lambda b, k, bx: (b, k, 0, 0, 0)),
        ],
    )
    ry = jnp.asarray(_RY)
    ry2b = jnp.concatenate([ry, ry], axis=1).astype(jnp.bfloat16)
    cxb = jnp.asarray(_CX).astype(jnp.bfloat16)
    masks, = pl.pallas_call(
        _mask_body,
        grid_spec=grid_spec,
        out_shape=[
            jax.ShapeDtypeStruct((_BS, _K, _T, _OUT, _OUT), jnp.uint8),
        ],
    )(boxes, *([pred_masks_r] * _G), ry2b, cxb)

    return (scores, labels, refs, masks)


# speedup vs baseline: 7.8664x; 1.3413x over previous
import jax
import jax.numpy as jnp
import numpy as np
from jax.experimental import pallas as pl
from jax.experimental.pallas import tpu as pltpu

_BS = 2
_Q = 300
_QP = 304
_C = 42
_T = 4
_HW = 48
_K = 100
_G = 10
_FLAT = _Q * _C
_PAD_COLS = 128
_PAD_ROWS = 99
_OUT = 256
_CROP_H = 180
_CROP_W = 192


def _interp_matrices():
    i = np.arange(_OUT)
    r = np.floor((i + 0.5) * (_CROP_H / _OUT)).astype(np.int64)
    y = (r + 0.5) / 4.0 - 0.5
    j0 = np.floor(y).astype(np.int64)
    w1 = (y - j0).astype(np.float32)
    ry = np.zeros((_OUT, _HW), np.float32)
    np.add.at(ry, (i, np.clip(j0, 0, _HW - 1)), 1.0 - w1)
    np.add.at(ry, (i, np.clip(j0 + 1, 0, _HW - 1)), w1)

    j = np.arange(_OUT)
    c = np.floor((j + 0.5) * (_CROP_W / _OUT)).astype(np.int64)
    x = (c + 0.5) / 4.0 - 0.5
    c0 = np.floor(x).astype(np.int64)
    wx = (x - c0).astype(np.float32)
    cx = np.zeros((_HW, _OUT), np.float32)
    np.add.at(cx, (np.clip(c0, 0, _HW - 1), j), 1.0 - wx)
    np.add.at(cx, (np.clip(c0 + 1, 0, _HW - 1), j), wx)
    return ry, cx


_RY, _CX = _interp_matrices()


def _topk_body(prob_ref, refs_ref, scores_ref, boxes_ref, labels_ref,
               refs_out_ref):
    rows = jax.lax.broadcasted_iota(jnp.int32, (_PAD_ROWS, _PAD_COLS), 0)
    cols = jax.lax.broadcasted_iota(jnp.int32, (_PAD_ROWS, _PAD_COLS), 1)
    flat = rows * _PAD_COLS + cols
    lanes = jax.lax.broadcasted_iota(jnp.int32, (1, _PAD_COLS), 1)

    def body(k, carry):
        p0, p1, sv0, sv1, iv0, iv1 = carry
        m0 = jnp.max(p0)
        m1 = jnp.max(p1)
        idx0 = jnp.min(jnp.where(p0 == m0, flat, jnp.int32(1 << 30)))
        idx1 = jnp.min(jnp.where(p1 == m1, flat, jnp.int32(1 << 30)))
        sv0 = jnp.where(lanes == k, m0, sv0)
        sv1 = jnp.where(lanes == k, m1, sv1)
        iv0 = jnp.where(lanes == k, idx0, iv0)
        iv1 = jnp.where(lanes == k, idx1, iv1)
        p0 = jnp.where(flat == idx0, jnp.float32(-1.0), p0)
        p1 = jnp.where(flat == idx1, jnp.float32(-1.0), p1)
        return p0, p1, sv0, sv1, iv0, iv1

    zf = jnp.zeros((1, _PAD_COLS), jnp.float32)
    zi = jnp.zeros((1, _PAD_COLS), jnp.int32)
    init = (prob_ref[0], prob_ref[1], zf, zf, zi, zi)
    _, _, sv0, sv1, iv0, iv1 = jax.lax.fori_loop(0, _K, body, init)

    qrow = jax.lax.broadcasted_iota(jnp.int32, (_QP, _PAD_COLS), 0)
    for b, (sv, iv) in enumerate(((sv0, iv0), (sv1, iv1))):
        bx = iv // _C
        scores_ref[b] = sv
        boxes_ref[b] = bx
        labels_ref[b] = iv % _C
        oht = (qrow == jnp.broadcast_to(bx, (_QP, _PAD_COLS))).astype(
            jnp.float32)
        refs_out_ref[b] = jax.lax.dot_general(
            oht, refs_ref[b], (((0,), (0,)), ((), ())),
            preferred_element_type=jnp.float32)


_KP = 104


def _gather_body(boxes_ref, pm_ref, gsel_ref):
    bxrow = boxes_ref[0].astype(jnp.float32)
    bxcol = jnp.transpose(bxrow)[:_KP]
    qiota = jax.lax.broadcasted_iota(
        jnp.int32, (_KP, _Q), 1).astype(jnp.float32)
    oh = (jnp.abs(qiota - bxcol) < 0.5).astype(jnp.bfloat16)
    nt = (((1,), (1,)), ((), ()))
    for h in range(_HW):
        x = pm_ref[0, 0, h]
        hi = x.astype(jnp.bfloat16)
        lo = (x - hi.astype(jnp.float32)).astype(jnp.bfloat16)
        o = (jax.lax.dot_general(oh, hi, nt, preferred_element_type=jnp.float32)
             + jax.lax.dot_general(oh, lo, nt,
                                   preferred_element_type=jnp.float32))
        gsel_ref[0, :, 0, h, :] = o


def _mask_body(*args):
    mrefs = args[:_G]
    ry2_ref, cx_ref, masks_out_ref = args[_G], args[_G + 1], args[_G + 2]
    ry2 = ry2_ref[...]
    cx = cx_ref[...]
    for g, mref in enumerate(mrefs):
        m2 = mref[0, 0]
        hi = m2.astype(jnp.bfloat16)
        lo = (m2 - hi.astype(jnp.float32)).astype(jnp.bfloat16)
        a1 = jnp.concatenate([hi, lo], axis=0)
        c2p = jnp.dot(a1, cx, preferred_element_type=jnp.float32)
        c2 = c2p[:_T * _HW] + c2p[_T * _HW:]
        c2h = c2.astype(jnp.bfloat16)
        c2l = (c2 - c2h.astype(jnp.float32)).astype(jnp.bfloat16)
        for t in range(_T):
            b2 = jnp.concatenate([c2h[t * _HW:(t + 1) * _HW],
                                  c2l[t * _HW:(t + 1) * _HW]], axis=0)
            o = jnp.dot(ry2, b2, preferred_element_type=jnp.float32)
            masks_out_ref[0, g, t] = (o > 0).astype(jnp.uint8)


@jax.jit
def kernel(pred_logits, pred_masks, reference_points, orig_target_sizes,
           max_target_sizes):
    del orig_target_sizes, max_target_sizes
    prob = jax.nn.sigmoid(pred_logits).reshape(_BS, _FLAT)
    prob = jnp.pad(prob, ((0, 0), (0, _PAD_ROWS * _PAD_COLS - _FLAT)),
                   constant_values=-1.0)
    prob = prob.reshape(_BS, _PAD_ROWS, _PAD_COLS)
    refs_pad = jnp.pad(reference_points, ((0, 0), (0, _QP - _Q), (0, 0)))

    scores3, boxes3, labels3, refsg = pl.pallas_call(
        _topk_body,
        grid=(1,),
        in_specs=[
            pl.BlockSpec((_BS, _PAD_ROWS, _PAD_COLS), lambda i: (0, 0, 0)),
            pl.BlockSpec((_BS, _QP, 4), lambda i: (0, 0, 0)),
        ],
        out_specs=[
            pl.BlockSpec((_BS, 1, _PAD_COLS), lambda i: (0, 0, 0)),
            pl.BlockSpec((_BS, 1, _PAD_COLS), lambda i: (0, 0, 0)),
            pl.BlockSpec((_BS, 1, _PAD_COLS), lambda i: (0, 0, 0)),
            pl.BlockSpec((_BS, _PAD_COLS, 4), lambda i: (0, 0, 0)),
        ],
        out_shape=[
            jax.ShapeDtypeStruct((_BS, 1, _PAD_COLS), jnp.float32),
            jax.ShapeDtypeStruct((_BS, 1, _PAD_COLS), jnp.int32),
            jax.ShapeDtypeStruct((_BS, 1, _PAD_COLS), jnp.int32),
            jax.ShapeDtypeStruct((_BS, _PAD_COLS, 4), jnp.float32),
        ],
    )(prob, refs_pad)

    scores = scores3.reshape(_BS, _PAD_COLS)[:, :_K]
    boxes = boxes3.reshape(_BS, _PAD_COLS)
    labels = labels3.reshape(_BS, _PAD_COLS)[:, :_K]
    refs = refsg[:, :_K, :]

    pmt = jnp.transpose(pred_masks, (0, 2, 3, 4, 1))
    gsel, = pl.pallas_call(
        _gather_body,
        grid=(_BS, _T),
        in_specs=[
            pl.BlockSpec((1, 1, _PAD_COLS), lambda b, t: (b, 0, 0)),
            pl.BlockSpec((1, 1, _HW, _HW, _Q), lambda b, t: (b, t, 0, 0, 0)),
        ],
        out_specs=[
            pl.BlockSpec((1, _KP, 1, _HW, _HW), lambda b, t: (b, 0, t, 0, 0)),
        ],
        out_shape=[
            jax.ShapeDtypeStruct((_BS, _KP, _T, _HW, _HW), jnp.float32),
        ],
    )(boxes3, pmt)

    gsel_r = gsel.reshape(_BS, _KP, _T * _HW, _HW)
    ry = jnp.asarray(_RY)
    ry2b = jnp.concatenate([ry, ry], axis=1).astype(jnp.bfloat16)
    cxb = jnp.asarray(_CX).astype(jnp.bfloat16)
    masks, = pl.pallas_call(
        _mask_body,
        grid=(_BS, _K // _G),
        in_specs=[
            pl.BlockSpec((1, 1, _T * _HW, _HW),
                         (lambda b, k, g=g: (b, _G * k + g, 0, 0)))
            for g in range(_G)
        ] + [
            pl.BlockSpec((_OUT, 2 * _HW), lambda b, k: (0, 0)),
            pl.BlockSpec((_HW, _OUT), lambda b, k: (0, 0)),
        ],
        out_specs=[
            pl.BlockSpec((1, _G, _T, _OUT, _OUT), lambda b, k: (b, k, 0, 0, 0)),
        ],
        out_shape=[
            jax.ShapeDtypeStruct((_BS, _K, _T, _OUT, _OUT), jnp.uint8),
        ],
    )(*([gsel_r] * _G), ry2b, cxb)

    return (scores, labels, refs, masks)


# speedup vs baseline: 7.8668x; 1.0000x over previous
import jax
import jax.numpy as jnp
import numpy as np
from jax.experimental import pallas as pl

_BS = 2
_Q = 300
_QP = 304
_C = 42
_T = 4
_HW = 48
_K = 100
_G = 10
_FLAT = _Q * _C
_PAD_COLS = 128
_PAD_ROWS = 99
_OUT = 256
_CROP_H = 180
_CROP_W = 192


def _interp_matrices():
    i = np.arange(_OUT)
    r = np.floor((i + 0.5) * (_CROP_H / _OUT)).astype(np.int64)
    y = (r + 0.5) / 4.0 - 0.5
    j0 = np.floor(y).astype(np.int64)
    w1 = (y - j0).astype(np.float32)
    ry = np.zeros((_OUT, _HW), np.float32)
    np.add.at(ry, (i, np.clip(j0, 0, _HW - 1)), 1.0 - w1)
    np.add.at(ry, (i, np.clip(j0 + 1, 0, _HW - 1)), w1)

    j = np.arange(_OUT)
    c = np.floor((j + 0.5) * (_CROP_W / _OUT)).astype(np.int64)
    x = (c + 0.5) / 4.0 - 0.5
    c0 = np.floor(x).astype(np.int64)
    wx = (x - c0).astype(np.float32)
    cx = np.zeros((_HW, _OUT), np.float32)
    np.add.at(cx, (np.clip(c0, 0, _HW - 1), j), 1.0 - wx)
    np.add.at(cx, (np.clip(c0 + 1, 0, _HW - 1), j), wx)
    return ry, cx


_RY, _CX = _interp_matrices()


def _topk_body(prob_ref, refs_ref, scores_ref, boxes_ref, labels_ref,
               refs_out_ref):
    rows = jax.lax.broadcasted_iota(jnp.int32, (_PAD_ROWS, _PAD_COLS), 0)
    cols = jax.lax.broadcasted_iota(jnp.int32, (_PAD_ROWS, _PAD_COLS), 1)
    flat = rows * _PAD_COLS + cols
    lanes = jax.lax.broadcasted_iota(jnp.int32, (1, _PAD_COLS), 1)

    def body(k, carry):
        p0, p1, sv0, sv1, iv0, iv1 = carry
        m0 = jnp.max(p0)
        m1 = jnp.max(p1)
        idx0 = jnp.min(jnp.where(p0 == m0, flat, jnp.int32(1 << 30)))
        idx1 = jnp.min(jnp.where(p1 == m1, flat, jnp.int32(1 << 30)))
        sv0 = jnp.where(lanes == k, m0, sv0)
        sv1 = jnp.where(lanes == k, m1, sv1)
        iv0 = jnp.where(lanes == k, idx0, iv0)
        iv1 = jnp.where(lanes == k, idx1, iv1)
        p0 = jnp.where(flat == idx0, jnp.float32(-1.0), p0)
        p1 = jnp.where(flat == idx1, jnp.float32(-1.0), p1)
        return p0, p1, sv0, sv1, iv0, iv1

    zf = jnp.zeros((1, _PAD_COLS), jnp.float32)
    zi = jnp.zeros((1, _PAD_COLS), jnp.int32)
    init = (prob_ref[0], prob_ref[1], zf, zf, zi, zi)
    _, _, sv0, sv1, iv0, iv1 = jax.lax.fori_loop(0, _K, body, init)

    qrow = jax.lax.broadcasted_iota(jnp.int32, (_QP, _PAD_COLS), 0)
    for b, (sv, iv) in enumerate(((sv0, iv0), (sv1, iv1))):
        bx = iv // _C
        scores_ref[b] = sv
        boxes_ref[b] = bx
        labels_ref[b] = iv % _C
        oht = (qrow == jnp.broadcast_to(bx, (_QP, _PAD_COLS))).astype(
            jnp.float32)
        refs_out_ref[b] = jax.lax.dot_general(
            oht, refs_ref[b], (((0,), (0,)), ((), ())),
            preferred_element_type=jnp.float32)


_KP = 104


def _gather_body(boxes_ref, pm_ref, gsel_ref):
    bxrow = boxes_ref[0].astype(jnp.float32)
    bxcol = jnp.transpose(bxrow)[:_KP]
    qiota = jax.lax.broadcasted_iota(
        jnp.int32, (_KP, _Q), 1).astype(jnp.float32)
    oh = (jnp.abs(qiota - bxcol) < 0.5).astype(jnp.bfloat16)
    nt = (((1,), (1,)), ((), ()))
    for h in range(_HW):
        x = pm_ref[0, 0, h]
        hi = x.astype(jnp.bfloat16)
        lo = (x - hi.astype(jnp.float32)).astype(jnp.bfloat16)
        o = (jax.lax.dot_general(oh, hi, nt, preferred_element_type=jnp.float32)
             + jax.lax.dot_general(oh, lo, nt,
                                   preferred_element_type=jnp.float32))
        gsel_ref[0, :, 0, h, :] = o


def _mask_body(*args):
    mrefs = args[:_G]
    ry2_ref, cx_ref, masks_out_ref = args[_G], args[_G + 1], args[_G + 2]
    ry2 = ry2_ref[...]
    cx = cx_ref[...]
    for g, mref in enumerate(mrefs):
        m2 = mref[0, 0]
        hi = m2.astype(jnp.bfloat16)
        lo = (m2 - hi.astype(jnp.float32)).astype(jnp.bfloat16)
        a1 = jnp.concatenate([hi, lo], axis=0)
        c2p = jnp.dot(a1, cx, preferred_element_type=jnp.float32)
        c2 = c2p[:_T * _HW] + c2p[_T * _HW:]
        c2h = c2.astype(jnp.bfloat16)
        c2l = (c2 - c2h.astype(jnp.float32)).astype(jnp.bfloat16)
        for t in range(_T):
            b2 = jnp.concatenate([c2h[t * _HW:(t + 1) * _HW],
                                  c2l[t * _HW:(t + 1) * _HW]], axis=0)
            o = jnp.dot(ry2, b2, preferred_element_type=jnp.float32)
            masks_out_ref[0, g, t] = (o > 0).astype(jnp.uint8)


@jax.jit
def kernel(pred_logits, pred_masks, reference_points, orig_target_sizes,
           max_target_sizes):
    del orig_target_sizes, max_target_sizes
    prob = jax.nn.sigmoid(pred_logits).reshape(_BS, _FLAT)
    prob = jnp.pad(prob, ((0, 0), (0, _PAD_ROWS * _PAD_COLS - _FLAT)),
                   constant_values=-1.0)
    prob = prob.reshape(_BS, _PAD_ROWS, _PAD_COLS)
    refs_pad = jnp.pad(reference_points, ((0, 0), (0, _QP - _Q), (0, 0)))

    scores3, boxes3, labels3, refsg = pl.pallas_call(
        _topk_body,
        grid=(1,),
        in_specs=[
            pl.BlockSpec((_BS, _PAD_ROWS, _PAD_COLS), lambda i: (0, 0, 0)),
            pl.BlockSpec((_BS, _QP, 4), lambda i: (0, 0, 0)),
        ],
        out_specs=[
            pl.BlockSpec((_BS, 1, _PAD_COLS), lambda i: (0, 0, 0)),
            pl.BlockSpec((_BS, 1, _PAD_COLS), lambda i: (0, 0, 0)),
            pl.BlockSpec((_BS, 1, _PAD_COLS), lambda i: (0, 0, 0)),
            pl.BlockSpec((_BS, _PAD_COLS, 4), lambda i: (0, 0, 0)),
        ],
        out_shape=[
            jax.ShapeDtypeStruct((_BS, 1, _PAD_COLS), jnp.float32),
            jax.ShapeDtypeStruct((_BS, 1, _PAD_COLS), jnp.int32),
            jax.ShapeDtypeStruct((_BS, 1, _PAD_COLS), jnp.int32),
            jax.ShapeDtypeStruct((_BS, _PAD_COLS, 4), jnp.float32),
        ],
    )(prob, refs_pad)

    scores = scores3.reshape(_BS, _PAD_COLS)[:, :_K]
    labels = labels3.reshape(_BS, _PAD_COLS)[:, :_K]
    refs = refsg[:, :_K, :]

    pmt = jnp.transpose(pred_masks, (0, 2, 3, 4, 1))
    gsel, = pl.pallas_call(
        _gather_body,
        grid=(_BS, _T),
        in_specs=[
            pl.BlockSpec((1, 1, _PAD_COLS), lambda b, t: (b, 0, 0)),
            pl.BlockSpec((1, 1, _HW, _HW, _Q), lambda b, t: (b, t, 0, 0, 0)),
        ],
        out_specs=[
            pl.BlockSpec((1, _KP, 1, _HW, _HW), lambda b, t: (b, 0, t, 0, 0)),
        ],
        out_shape=[
            jax.ShapeDtypeStruct((_BS, _KP, _T, _HW, _HW), jnp.float32),
        ],
    )(boxes3, pmt)

    gsel_r = gsel.reshape(_BS, _KP, _T * _HW, _HW)
    ry = jnp.asarray(_RY)
    ry2b = jnp.concatenate([ry, ry], axis=1).astype(jnp.bfloat16)
    cxb = jnp.asarray(_CX).astype(jnp.bfloat16)
    masks, = pl.pallas_call(
        _mask_body,
        grid=(_BS, _K // _G),
        in_specs=[
            pl.BlockSpec((1, 1, _T * _HW, _HW),
                         (lambda b, k, g=g: (b, _G * k + g, 0, 0)))
            for g in range(_G)
        ] + [
            pl.BlockSpec((_OUT, 2 * _HW), lambda b, k: (0, 0)),
            pl.BlockSpec((_HW, _OUT), lambda b, k: (0, 0)),
        ],
        out_specs=[
            pl.BlockSpec((1, _G, _T, _OUT, _OUT), lambda b, k: (b, k, 0, 0, 0)),
        ],
        out_shape=[
            jax.ShapeDtypeStruct((_BS, _K, _T, _OUT, _OUT), jnp.uint8),
        ],
    )(*([gsel_r] * _G), ry2b, cxb)

    return (scores, labels, refs, masks)
